# Initial kernel scaffold; baseline (speedup 1.0000x reference)
import jax
import jax.numpy as jnp
from jax.experimental import pallas as pl


def kernel(input_features, flat_indices, batch_size, spatial):
    raise NotImplementedError("write your pallas kernel here")


# speedup vs baseline: 1.1317x; 1.1317x over previous
import functools

import jax
import jax.numpy as jnp
from jax import lax
from jax.experimental import pallas as pl
from jax.experimental.pallas import tpu as pltpu
from jax.experimental.pallas import tpu_sc as plsc

N_ACTIVE = 100000
C = 64
B = 2
S3 = 64 * 64 * 64
TOTAL = B * S3

NC = 2
NS = 16
CHUNK = 128
K = 49
PER_TILE = K * CHUNK
N_PAD = NS * PER_TILE
CH_PER_SC = C // NC
SLICE = TOTAL // NS

_mesh = plsc.VectorSubcoreMesh(
    core_axis_name="c", subcore_axis_name="s", num_cores=NC, num_subcores=NS
)


@functools.partial(
    pl.kernel,
    out_type=jax.ShapeDtypeStruct((B, C, S3), jnp.float32),
    mesh=_mesh,
    scratch_types=[
        pltpu.VMEM((K, CHUNK), jnp.int32),
        pltpu.VMEM((K, CHUNK), jnp.float32),
        pltpu.VMEM((SLICE,), jnp.float32),
        pltpu.VMEM_SHARED((TOTAL,), jnp.float32),
        pltpu.SemaphoreType.DMA,
    ],
)
def _sc_scatter(featT_hbm, idx_hbm, out_hbm, idx_v, vals_v, zbuf, acc, sem):
    cid = lax.axis_index("c")
    sid = lax.axis_index("s")
    my = pl.ds(sid * SLICE, SLICE)
    b_out = sid // (NS // B)
    s_out = (sid % (NS // B)) * SLICE

    zero16 = jnp.zeros((16,), jnp.float32)

    def _zb(i, carry):
        zbuf[pl.ds(i * 16, 16)] = zero16
        return carry

    lax.fori_loop(0, SLICE // 16, _zb, 0)

    pltpu.sync_copy(idx_hbm.at[sid], idx_v)
    pltpu.sync_copy(zbuf, acc.at[my])
    plsc.subcore_barrier()

    def _pass(p, carry):
        ch = cid * CH_PER_SC + p
        pltpu.sync_copy(featT_hbm.at[ch, sid], vals_v)

        def _fire(j, c2):
            pltpu.async_copy(vals_v.at[j], acc.at[idx_v.at[j]], sem, add=True)
            return c2

        lax.fori_loop(0, K, _fire, 0)
        pltpu.make_async_copy(featT_hbm.at[ch, sid], vals_v, sem).wait()
        plsc.subcore_barrier()

        pltpu.sync_copy(acc.at[my], out_hbm.at[b_out, ch, pl.ds(s_out, SLICE)])
        pltpu.sync_copy(zbuf, acc.at[my])
        plsc.subcore_barrier()
        return carry

    lax.fori_loop(0, CH_PER_SC, _pass, 0)


def kernel(input_features, flat_indices, batch_size, spatial):
    n = input_features.shape[0]
    idx = flat_indices.astype(jnp.int32)
    idx = jnp.pad(idx, (0, N_PAD - n)).reshape(NS, K, CHUNK)
    featt = (
        jnp.pad(input_features, ((0, N_PAD - n), (0, 0)))
        .T.reshape(C, NS, K, CHUNK)
    )
    out = _sc_scatter(featt, idx)
    return out.reshape(B, C, 64, 64, 64)
